# Initial kernel scaffold; baseline (speedup 1.0000x reference)
import jax
import jax.numpy as jnp
from jax.experimental import pallas as pl


def kernel(prot_x, mol_feat, segment_ids, W_prot, b_prot, W_ffn, b_ffn):
    raise NotImplementedError("write your pallas kernel here")


# speedup vs baseline: 4.0554x; 4.0554x over previous
import functools

import jax
import jax.numpy as jnp
from jax import lax
from jax.experimental import pallas as pl

B = 16
N = 32768
D = 1280
H = 128

ROWS = 2048
NBLK = N // ROWS
IDS_R = ROWS // 128


def _seg_kernel(x_ref, ids_ref, mol_ref, wp_ref, bp_ref, wf_ref, bf_ref,
                out_ref, pgf_ref, sum_ref, cnt_ref):
    i = pl.program_id(0)

    @pl.when(i == 0)
    def _init():
        sum_ref[...] = jnp.zeros_like(sum_ref)
        cnt_ref[...] = jnp.zeros_like(cnt_ref)

    seg = ids_ref[...].reshape(1, ROWS)
    bidx = lax.broadcasted_iota(jnp.int32, (B, ROWS), 0)
    onehot = (bidx == seg).astype(jnp.float32)
    sum_ref[...] += jnp.dot(onehot, x_ref[...],
                            preferred_element_type=jnp.float32)
    cnt_ref[...] += jnp.sum(onehot, axis=1, keepdims=True)

    @pl.when(i == NBLK - 1)
    def _head():
        cnt = cnt_ref[:, :1]
        mean = sum_ref[...] / jnp.maximum(cnt, 1.0)
        nonempty = (cnt > 0.0).astype(jnp.float32)
        pgf = (jnp.dot(mean, wp_ref[...],
                       preferred_element_type=jnp.float32)
               + bp_ref[...] * nonempty)
        pgf_ref[...] = pgf
        w_mol = wf_ref[:, :H]
        w_pgf = wf_ref[:, H:]
        out_ref[...] = (jnp.sum(mol_ref[...] * w_mol, axis=1, keepdims=True)
                        + jnp.sum(pgf * w_pgf, axis=1, keepdims=True)
                        + bf_ref[0, 0])


@jax.jit
def _run(prot_x, mol_feat, ids2d, W_prot, b_prot2d, W_ffn_t, b_ffn2d):
    out, pgf, _, _ = pl.pallas_call(
        _seg_kernel,
        grid=(NBLK,),
        in_specs=[
            pl.BlockSpec((ROWS, D), lambda i: (i, 0)),
            pl.BlockSpec((IDS_R, 128), lambda i: (i, 0)),
            pl.BlockSpec((B, H), lambda i: (0, 0)),
            pl.BlockSpec((D, H), lambda i: (0, 0)),
            pl.BlockSpec((1, H), lambda i: (0, 0)),
            pl.BlockSpec((1, 2 * H), lambda i: (0, 0)),
            pl.BlockSpec((1, 1), lambda i: (0, 0)),
        ],
        out_specs=[
            pl.BlockSpec((B, 1), lambda i: (0, 0)),
            pl.BlockSpec((B, H), lambda i: (0, 0)),
            pl.BlockSpec((B, D), lambda i: (0, 0)),
            pl.BlockSpec((B, 1), lambda i: (0, 0)),
        ],
        out_shape=[
            jax.ShapeDtypeStruct((B, 1), jnp.float32),
            jax.ShapeDtypeStruct((B, H), jnp.float32),
            jax.ShapeDtypeStruct((B, D), jnp.float32),
            jax.ShapeDtypeStruct((B, 1), jnp.float32),
        ],
    )(prot_x, ids2d, mol_feat, W_prot, b_prot2d, W_ffn_t, b_ffn2d)
    return out, pgf


def kernel(prot_x, mol_feat, segment_ids, W_prot, b_prot, W_ffn, b_ffn):
    ids2d = segment_ids.astype(jnp.int32).reshape(N // 128, 128)
    out, pgf = _run(prot_x, mol_feat, ids2d, W_prot,
                    b_prot.reshape(1, H), W_ffn.reshape(1, 2 * H).astype(jnp.float32),
                    b_ffn.reshape(1, 1))
    return (out, mol_feat, pgf)
